# baseline (speedup 1.0000x reference)
import functools

import jax
import jax.numpy as jnp
from jax import lax
from jax.experimental import pallas as pl
from jax.experimental.pallas import tpu as pltpu
from jax.experimental.pallas import tpu_sc as plsc

_LANES = 16
_EPS = 1e-5


def _rsqrt16(v16):
    bits = plsc.bitcast(v16, jnp.int32)
    y = plsc.bitcast(jnp.int32(0x5F3759DF) - (bits >> 1), jnp.float32)
    half = v16 * 0.5
    for _ in range(3):
        y = y * (1.5 - half * y * y)
    return y


def _build_sc_call(N, S, V, D, MAXP):
    info = plsc.get_sparse_core_info()
    NC, NS = info.num_cores, info.num_subcores
    NW = NC * NS
    R = N // NW
    K = 16
    G = R // K
    NVEC = D // _LANES
    assert N % NW == 0 and R % K == 0 and D % _LANES == 0 and S % R == 0

    mesh = plsc.VectorSubcoreMesh(core_axis_name="c", subcore_axis_name="s")

    @functools.partial(
        pl.kernel,
        mesh=mesh,
        out_type=jax.ShapeDtypeStruct((N, D), jnp.float32),
        compiler_params=pltpu.CompilerParams(needs_layout_passes=False),
        scratch_types=[
            pltpu.VMEM((K,), jnp.int32),
            pltpu.VMEM((K, D), jnp.float32),
            pltpu.VMEM((K, D), jnp.float32),
            pltpu.VMEM((D,), jnp.float32),
            pltpu.VMEM((D,), jnp.float32),
            pltpu.SemaphoreType.DMA,
        ],
    )
    def emb_kernel(ids_hbm, word_hbm, pos_hbm, gamma_hbm, beta_hbm, out_hbm,
                   idx_v, rows_v, pos_v, gamma_v, beta_v, gsem):
        wid = lax.axis_index("s") * NC + lax.axis_index("c")
        base = wid * R
        pos0 = base % S
        pltpu.sync_copy(gamma_hbm, gamma_v)
        pltpu.sync_copy(beta_hbm, beta_v)

        def chunk_body(g, carry):
            row0 = base + g * K
            pltpu.sync_copy(ids_hbm.at[pl.ds(row0, K)], idx_v)
            cp = pltpu.async_copy(word_hbm.at[idx_v], rows_v, gsem)
            pltpu.sync_copy(pos_hbm.at[pl.ds(pos0 + g * K, K)], pos_v)
            cp.wait()

            def row_body(r, carry2):
                acc_s = jnp.zeros((_LANES,), jnp.float32)
                acc_q = jnp.zeros((_LANES,), jnp.float32)
                for j in range(NVEC):
                    sl = pl.ds(j * _LANES, _LANES)
                    x = rows_v[r, sl] + pos_v[r, sl]
                    rows_v[r, sl] = x
                    acc_s = acc_s + x
                    acc_q = acc_q + x * x
                mean = jnp.sum(acc_s) * (1.0 / D)
                var = jnp.sum(acc_q) * (1.0 / D) - mean * mean
                rstd = _rsqrt16(jnp.full((_LANES,), var + _EPS, jnp.float32))
                m16 = jnp.full((_LANES,), mean, jnp.float32)
                for j in range(NVEC):
                    sl = pl.ds(j * _LANES, _LANES)
                    x = rows_v[r, sl]
                    rows_v[r, sl] = (x - m16) * rstd * gamma_v[sl] + beta_v[sl]
                return carry2

            lax.fori_loop(0, K, row_body, 0)
            pltpu.sync_copy(rows_v, out_hbm.at[pl.ds(row0, K)])
            return carry

        lax.fori_loop(0, G, chunk_body, 0)

    return emb_kernel


def kernel(input_ids, word_table, pos_table, ln_gamma, ln_beta):
    B, S = input_ids.shape
    V, D = word_table.shape
    MAXP = pos_table.shape[0]
    N = B * S
    ids_flat = input_ids.reshape(N).astype(jnp.int32)
    call = _build_sc_call(N, S, V, D, MAXP)
    out = call(ids_flat, word_table, pos_table, ln_gamma, ln_beta)
    return out.reshape(B, S, D)


# speedup vs baseline: 1.3738x; 1.3738x over previous
import functools

import jax
import jax.numpy as jnp
from jax import lax
from jax.experimental import pallas as pl
from jax.experimental.pallas import tpu as pltpu
from jax.experimental.pallas import tpu_sc as plsc

_LANES = 16
_EPS = 1e-5


def _rsqrt16(v16):
    bits = plsc.bitcast(v16, jnp.int32)
    y = plsc.bitcast(jnp.int32(0x5F3759DF) - (bits >> 1), jnp.float32)
    half = v16 * 0.5
    for _ in range(3):
        y = y * (1.5 - half * y * y)
    return y


def _build_sc_call(N, S, V, D, MAXP):
    info = plsc.get_sparse_core_info()
    NC, NS = info.num_cores, info.num_subcores
    NW = NC * NS
    R = N // NW
    K = 16
    G = R // K
    NVEC = D // _LANES
    assert N % NW == 0 and R % K == 0 and D % _LANES == 0 and S % R == 0

    mesh = plsc.VectorSubcoreMesh(core_axis_name="c", subcore_axis_name="s")

    @functools.partial(
        pl.kernel,
        mesh=mesh,
        out_type=jax.ShapeDtypeStruct((N, D), jnp.float32),
        compiler_params=pltpu.CompilerParams(needs_layout_passes=False),
        scratch_types=[
            pltpu.VMEM((G, K), jnp.int32),
            pltpu.VMEM((2, K, D), jnp.float32),
            pltpu.VMEM((2, K, D), jnp.float32),
            pltpu.VMEM((2, K, D), jnp.float32),
            pltpu.VMEM((D,), jnp.float32),
            pltpu.VMEM((D,), jnp.float32),
            pltpu.SemaphoreType.DMA((2,)),
            pltpu.SemaphoreType.DMA((2,)),
            pltpu.SemaphoreType.DMA((2,)),
        ],
    )
    def emb_kernel(ids_hbm, word_hbm, pos_hbm, gamma_hbm, beta_hbm, out_hbm,
                   ids_v, rows_v, pos_v, outb_v, gamma_v, beta_v,
                   gsem, psem, osem):
        wid = lax.axis_index("s") * NC + lax.axis_index("c")
        base = wid * R
        pos0 = base % S
        crow0 = wid * G
        pltpu.sync_copy(gamma_hbm, gamma_v)
        pltpu.sync_copy(beta_hbm, beta_v)
        pltpu.sync_copy(ids_hbm.at[pl.ds(crow0, G)], ids_v)

        def start_fetch(g, b):
            pltpu.async_copy(word_hbm.at[ids_v.at[g]], rows_v.at[b],
                             gsem.at[b])
            pltpu.async_copy(pos_hbm.at[pl.ds(pos0 + g * K, K)], pos_v.at[b],
                             psem.at[b])

        start_fetch(0, 0)

        def chunk_body(g, carry):
            b = lax.rem(g, 2)
            nb = 1 - b

            @pl.when(g + 1 < G)
            def _():
                start_fetch(g + 1, nb)

            pltpu.make_async_copy(word_hbm.at[ids_v.at[g]], rows_v.at[b],
                                  gsem.at[b]).wait()
            pltpu.make_async_copy(pos_hbm.at[pl.ds(pos0 + g * K, K)],
                                  pos_v.at[b], psem.at[b]).wait()

            @pl.when(g >= 2)
            def _():
                pltpu.make_async_copy(
                    outb_v.at[b], out_hbm.at[pl.ds(base, K)], osem.at[b]
                ).wait()

            def row_body(r, carry2):
                acc_s = jnp.zeros((_LANES,), jnp.float32)
                acc_q = jnp.zeros((_LANES,), jnp.float32)
                for j in range(NVEC):
                    sl = pl.ds(j * _LANES, _LANES)
                    x = rows_v[b, r, sl] + pos_v[b, r, sl]
                    rows_v[b, r, sl] = x
                    acc_s = acc_s + x
                    acc_q = acc_q + x * x
                mean = jnp.sum(acc_s) * (1.0 / D)
                var = jnp.sum(acc_q) * (1.0 / D) - mean * mean
                rstd = _rsqrt16(jnp.full((_LANES,), var + _EPS, jnp.float32))
                m16 = jnp.full((_LANES,), mean, jnp.float32)
                for j in range(NVEC):
                    sl = pl.ds(j * _LANES, _LANES)
                    x = rows_v[b, r, sl]
                    outb_v[b, r, sl] = (x - m16) * rstd * gamma_v[sl] + beta_v[sl]
                return carry2

            lax.fori_loop(0, K, row_body, 0)
            pltpu.async_copy(outb_v.at[b],
                             out_hbm.at[pl.ds(base + g * K, K)], osem.at[b])
            return carry

        lax.fori_loop(0, G, chunk_body, 0)

        for tail in (G - 2, G - 1):
            b = tail % 2
            pltpu.make_async_copy(
                outb_v.at[b], out_hbm.at[pl.ds(base, K)], osem.at[b]
            ).wait()

    return emb_kernel


def kernel(input_ids, word_table, pos_table, ln_gamma, ln_beta):
    B, S = input_ids.shape
    V, D = word_table.shape
    MAXP = pos_table.shape[0]
    N = B * S
    K = 16
    ids_2d = input_ids.reshape(N // K, K).astype(jnp.int32)
    call = _build_sc_call(N, S, V, D, MAXP)
    out = call(ids_2d, word_table, pos_table, ln_gamma, ln_beta)
    return out.reshape(B, S, D)
